# Initial kernel scaffold; baseline (speedup 1.0000x reference)
import functools

import jax
import jax.numpy as jnp
from jax import lax
from jax.experimental import pallas as pl
from jax.experimental.pallas import tpu as pltpu
from jax.experimental.pallas import tpu_sc as plsc

D = 128
N_NODE = 10000
PAD_N = 10240
NS = 16
RPT = PAD_N // NS
CH = 128
E = 320000
T_CH = -(-E // (NS * CH))
EPT = T_CH * CH
E_PAD = EPT * NS
CW = 16

_mesh = plsc.VectorSubcoreMesh(core_axis_name="c", subcore_axis_name="s")


@functools.partial(
    pl.kernel,
    out_type=[
        jax.ShapeDtypeStruct((PAD_N, D), jnp.float32),
        jax.ShapeDtypeStruct((PAD_N, D), jnp.float32),
    ],
    mesh=_mesh,
    scratch_types=[
        pltpu.VMEM((CH,), jnp.int32),
        pltpu.VMEM((CH,), jnp.int32),
        pltpu.VMEM((CH, D), jnp.float32),
        pltpu.VMEM((CH, CW), jnp.float32),
        pltpu.VMEM((RPT, D), jnp.float32),
        pltpu.VMEM((RPT, CW), jnp.float32),
        pltpu.VMEM_SHARED((PAD_N, D), jnp.float32),
        pltpu.VMEM_SHARED((PAD_N, CW), jnp.float32),
        pltpu.SemaphoreType.DMA,
    ],
)
def _sc_aggregate(src_ui, dst_ui, src_iu, dst_iu, x_user, x_item,
                  zeros_f, zeros_c, ones_c,
                  mean_ui, mean_iu,
                  idx_s, idx_d, rows, ones_v, obuf, cbuf, acc_sh, cnt_sh, sem):
    c = lax.axis_index("c")
    s = lax.axis_index("s")
    rbase = pl.multiple_of(s * RPT, 8)
    ebase = pl.multiple_of(s * EPT, 8)

    def run_type(src_hbm, dst_hbm, x_hbm, out_hbm):
        pltpu.sync_copy(zeros_f, obuf)
        pltpu.sync_copy(obuf, acc_sh.at[pl.ds(rbase, RPT)])
        pltpu.sync_copy(zeros_c, cbuf)
        pltpu.sync_copy(cbuf, cnt_sh.at[pl.ds(rbase, RPT)])
        pltpu.sync_copy(ones_c, ones_v)
        plsc.subcore_barrier()

        def chunk(i, carry):
            base = pl.multiple_of(ebase + i * CH, 8)
            pltpu.sync_copy(src_hbm.at[pl.ds(base, CH)], idx_s)
            pltpu.async_copy(x_hbm.at[idx_s], rows, sem).wait()
            pltpu.sync_copy(dst_hbm.at[pl.ds(base, CH)], idx_d)
            pltpu.sync_copy(rows, acc_sh.at[idx_d], add=True)
            pltpu.sync_copy(ones_v, cnt_sh.at[idx_d], add=True)
            return carry

        lax.fori_loop(0, T_CH, chunk, 0)
        plsc.subcore_barrier()

        pltpu.sync_copy(acc_sh.at[pl.ds(rbase, RPT)], obuf)
        pltpu.sync_copy(cnt_sh.at[pl.ds(rbase, RPT)], cbuf)

        def row(j, carry):
            inv = 1.0 / jnp.maximum(cbuf[j], 1.0)
            for k in range(D // 16):
                obuf[j, pl.ds(k * 16, 16)] = obuf[j, pl.ds(k * 16, 16)] * inv
            return carry

        lax.fori_loop(0, RPT, row, 0)
        pltpu.sync_copy(obuf, out_hbm.at[pl.ds(rbase, RPT)])

    @pl.when(c == 0)
    def _():
        run_type(src_ui, dst_ui, x_user, mean_ui)

    @pl.when(c == 1)
    def _():
        run_type(src_iu, dst_iu, x_item, mean_iu)


def _dense_body(mean_ref, x_ref, wr_ref, br_ref, wt_ref, o_ref):
    dn = (((1,), (1,)), ((), ()))
    o_ref[...] = (
        lax.dot_general(mean_ref[...], wr_ref[...], dn,
                        preferred_element_type=jnp.float32)
        + br_ref[...]
        + lax.dot_general(x_ref[...], wt_ref[...], dn,
                          preferred_element_type=jnp.float32)
    )


def _dense(mean_pad, x_dst, W_rel, b_rel, W_root):
    blk = 1000
    return pl.pallas_call(
        _dense_body,
        grid=(N_NODE // blk,),
        in_specs=[
            pl.BlockSpec((blk, D), lambda i: (i, 0)),
            pl.BlockSpec((blk, D), lambda i: (i, 0)),
            pl.BlockSpec((D, D), lambda i: (0, 0)),
            pl.BlockSpec((1, D), lambda i: (0, 0)),
            pl.BlockSpec((D, D), lambda i: (0, 0)),
        ],
        out_specs=pl.BlockSpec((blk, D), lambda i: (i, 0)),
        out_shape=jax.ShapeDtypeStruct((N_NODE, D), jnp.float32),
    )(mean_pad, x_dst, W_rel, b_rel.reshape(1, D), W_root)


def kernel(x_user, x_item, edge_index_user_item, edge_index_item_user,
           W_rel_ui, b_rel_ui, W_root_ui, W_rel_iu, b_rel_iu, W_root_iu):
    pad = E_PAD - E
    pad_src = jnp.zeros((pad,), jnp.int32)
    pad_dst = jnp.full((pad,), PAD_N - 1, jnp.int32)

    def split_pad(edge_index):
        src = jnp.concatenate([edge_index[0].astype(jnp.int32), pad_src])
        dst = jnp.concatenate([edge_index[1].astype(jnp.int32), pad_dst])
        return src, dst

    src_ui, dst_ui = split_pad(edge_index_user_item)
    src_iu, dst_iu = split_pad(edge_index_item_user)

    zeros_f = jnp.zeros((RPT, D), jnp.float32)
    zeros_c = jnp.zeros((RPT, CW), jnp.float32)
    ones_c = jnp.ones((CH, CW), jnp.float32)

    mean_ui, mean_iu = _sc_aggregate(src_ui, dst_ui, src_iu, dst_iu,
                                     x_user, x_item, zeros_f, zeros_c, ones_c)

    out_item = _dense(mean_ui, x_item, W_rel_ui, b_rel_ui, W_root_ui)
    out_user = _dense(mean_iu, x_user, W_rel_iu, b_rel_iu, W_root_iu)
    return (out_user, out_item)


# speedup vs baseline: 5.3908x; 5.3908x over previous
import functools

import jax
import jax.numpy as jnp
from jax import lax
from jax.experimental import pallas as pl
from jax.experimental.pallas import tpu as pltpu
from jax.experimental.pallas import tpu_sc as plsc

D = 128
N_NODE = 10000
PAD_N = 10240
NS = 16
RPT = PAD_N // NS
CH = 96
E = 320000
T_CH = -(-E // (NS * CH))
EPT = T_CH * CH
E_PAD = EPT * NS
FS = 32

_mesh = plsc.VectorSubcoreMesh(core_axis_name="c", subcore_axis_name="s")


@functools.partial(
    pl.kernel,
    out_type=[jax.ShapeDtypeStruct((2 * PAD_N, D), jnp.float32),
              jax.ShapeDtypeStruct((2, NS, PAD_N), jnp.float32)],
    mesh=_mesh,
    compiler_params=pltpu.CompilerParams(needs_layout_passes=False),
    scratch_types=[
        pltpu.VMEM((CH,), jnp.int32),
        pltpu.VMEM((CH,), jnp.int32),
        pltpu.VMEM((CH, D), jnp.float32),
        pltpu.VMEM((FS, D), jnp.float32),
        pltpu.VMEM((PAD_N,), jnp.float32),
        pltpu.VMEM((RPT,), jnp.float32),
        pltpu.VMEM((RPT,), jnp.float32),
        pltpu.VMEM_SHARED((PAD_N, D), jnp.float32),
        pltpu.SemaphoreType.DMA,
    ],
)
def _sc_aggregate(src_all, dst_all, x_all, zeros_f, zeros_n,
                  mean_all, cnt_pub,
                  idx_s, idx_d, rows, obuf, cnt_loc, tmp, csum, acc_sh, sem):
    c = lax.axis_index("c")
    s = lax.axis_index("s")
    rbase = pl.multiple_of(s * RPT, 8)
    obase = pl.multiple_of(c * PAD_N + s * RPT, 8)
    ebase = pl.multiple_of(c * E_PAD + s * EPT, 8)

    pltpu.sync_copy(zeros_f, obuf)

    def zero_slab(q, carry):
        rb = pl.multiple_of(rbase + q * FS, 8)
        pltpu.sync_copy(obuf, acc_sh.at[pl.ds(rb, FS)])
        return carry

    lax.fori_loop(0, RPT // FS, zero_slab, 0)
    pltpu.sync_copy(zeros_n, cnt_loc)
    plsc.subcore_barrier()

    ones = jnp.full((16,), 1.0, jnp.float32)

    def chunk(i, carry):
        base = pl.multiple_of(ebase + i * CH, 8)
        pltpu.sync_copy(src_all.at[pl.ds(base, CH)], idx_s)
        pltpu.async_copy(x_all.at[idx_s], rows, sem).wait()
        pltpu.sync_copy(dst_all.at[pl.ds(base, CH)], idx_d)
        pltpu.sync_copy(rows, acc_sh.at[idx_d], add=True)
        for k in range(CH // 16):
            v = idx_d[pl.ds(k * 16, 16)]
            plsc.addupdate_scatter(cnt_loc, [v], ones)
        return carry

    lax.fori_loop(0, T_CH, chunk, 0)
    pltpu.sync_copy(cnt_loc, cnt_pub.at[c, s])
    plsc.subcore_barrier()

    pltpu.sync_copy(cnt_pub.at[c, 0, pl.ds(rbase, RPT)], csum)
    for t in range(1, NS):
        pltpu.sync_copy(cnt_pub.at[c, t, pl.ds(rbase, RPT)], tmp)
        for m in range(RPT // 16):
            csum[pl.ds(m * 16, 16)] = (csum[pl.ds(m * 16, 16)]
                                       + tmp[pl.ds(m * 16, 16)])
    for m in range(RPT // 16):
        csum[pl.ds(m * 16, 16)] = 1.0 / jnp.maximum(csum[pl.ds(m * 16, 16)], 1.0)

    def fin_slab(q, carry):
        rb = pl.multiple_of(rbase + q * FS, 8)
        pltpu.sync_copy(acc_sh.at[pl.ds(rb, FS)], obuf)
        for j in range(FS):
            ridx = jnp.full((16,), q * FS + j, jnp.int32)
            inv = plsc.load_gather(csum, [ridx])
            for k in range(D // 16):
                obuf[j, pl.ds(k * 16, 16)] = obuf[j, pl.ds(k * 16, 16)] * inv
        ob = pl.multiple_of(obase + q * FS, 8)
        pltpu.sync_copy(obuf, mean_all.at[pl.ds(ob, FS)])
        return carry

    lax.fori_loop(0, RPT // FS, fin_slab, 0)


def _dense_body(mean_ref, x_ref, wr_ref, br_ref, wt_ref, o_ref):
    dn = (((1,), (1,)), ((), ()))
    o_ref[...] = (
        lax.dot_general(mean_ref[...], wr_ref[...], dn,
                        preferred_element_type=jnp.float32)
        + br_ref[...]
        + lax.dot_general(x_ref[...], wt_ref[...], dn,
                          preferred_element_type=jnp.float32)
    )


def _dense(mean, x_dst, W_rel, b_rel, W_root):
    blk = 1000
    return pl.pallas_call(
        _dense_body,
        grid=(N_NODE // blk,),
        in_specs=[
            pl.BlockSpec((blk, D), lambda i: (i, 0)),
            pl.BlockSpec((blk, D), lambda i: (i, 0)),
            pl.BlockSpec((D, D), lambda i: (0, 0)),
            pl.BlockSpec((1, D), lambda i: (0, 0)),
            pl.BlockSpec((D, D), lambda i: (0, 0)),
        ],
        out_specs=pl.BlockSpec((blk, D), lambda i: (i, 0)),
        out_shape=jax.ShapeDtypeStruct((N_NODE, D), jnp.float32),
    )(mean, x_dst, W_rel, b_rel.reshape(1, D), W_root)


def kernel(x_user, x_item, edge_index_user_item, edge_index_item_user,
           W_rel_ui, b_rel_ui, W_root_ui, W_rel_iu, b_rel_iu, W_root_iu):
    pad = E_PAD - E
    pad_dst = jnp.full((pad,), PAD_N - 1, jnp.int32)

    def pad_type(edge_index, src_off):
        src = jnp.concatenate([edge_index[0].astype(jnp.int32) + src_off,
                               jnp.full((pad,), src_off, jnp.int32)])
        dst = jnp.concatenate([edge_index[1].astype(jnp.int32), pad_dst])
        return src, dst

    src_ui, dst_ui = pad_type(edge_index_user_item, 0)
    src_iu, dst_iu = pad_type(edge_index_item_user, N_NODE)
    src_all = jnp.concatenate([src_ui, src_iu])
    dst_all = jnp.concatenate([dst_ui, dst_iu])
    x_all = jnp.concatenate([x_user, x_item])

    zeros_f = jnp.zeros((FS, D), jnp.float32)
    zeros_n = jnp.zeros((PAD_N,), jnp.float32)

    mean_all, _ = _sc_aggregate(src_all, dst_all, x_all, zeros_f, zeros_n)
    mean_ui = mean_all[:N_NODE]
    mean_iu = mean_all[PAD_N:PAD_N + N_NODE]

    out_item = _dense(mean_ui, x_item, W_rel_ui, b_rel_ui, W_root_ui)
    out_user = _dense(mean_iu, x_user, W_rel_iu, b_rel_iu, W_root_iu)
    return (out_user, out_item)
